# baseline (speedup 1.0000x reference)
import functools

import jax
import jax.numpy as jnp
from jax import lax
from jax.experimental import pallas as pl
from jax.experimental.pallas import tpu as pltpu
from jax.experimental.pallas import tpu_sc as plsc

N = 10000
E = 160000
F = 256
FH = 128
HOPS = 6
NC, NS, L = 2, 16, 16
CH = 128
NBUF = 2
NCH = 80
EPS = NCH * CH
EPAD = EPS * NS
NPAD = 10240
WRO = NPAD // NS
NBLK = WRO // CH
BLK2 = 64
NBLK2 = WRO // BLK2
RB = 1000

_mesh = plsc.VectorSubcoreMesh(
    core_axis_name="c", subcore_axis_name="s", num_cores=NC, num_subcores=NS)


def _fill(buf, rows, val):
    v = jnp.full((L,), val, jnp.float32)

    @pl.loop(0, rows)
    def _(r):
        for c in range(buf.shape[1] // L):
            buf[r, pl.ds(c * L, L)] = v


@functools.partial(
    pl.kernel,
    out_type=jax.ShapeDtypeStruct((NPAD, L), jnp.float32),
    mesh=_mesh,
    scratch_types=[
        pltpu.VMEM_SHARED((NPAD, L), jnp.float32),
        pltpu.VMEM((NCH, CH), jnp.int32),
        pltpu.VMEM((CH, L), jnp.float32),
    ],
)
def _deg_kernel(dst3_hbm, out_hbm, hist, didx, ones_v):
    c = lax.axis_index("c")
    s = lax.axis_index("s")

    @pl.when(c == 0)
    def _():
        _fill(ones_v, CH, 0.0)

        @pl.loop(0, NBLK)
        def _(t):
            pltpu.sync_copy(ones_v, hist.at[pl.ds(s * WRO + t * CH, CH)])

        _fill(ones_v, CH, 1.0)
        pltpu.sync_copy(dst3_hbm.at[s], didx)
        plsc.subcore_barrier()

        @pl.loop(0, NCH)
        def _(j):
            pltpu.sync_copy(ones_v, hist.at[didx.at[j]], add=True)

        plsc.subcore_barrier()
        pltpu.sync_copy(hist.at[pl.ds(s * WRO, WRO)],
                        out_hbm.at[pl.ds(s * WRO, WRO)])


GI = 16
NG = NCH // GI


@functools.partial(
    pl.kernel,
    out_type=jax.ShapeDtypeStruct((HOPS, NC, NPAD, FH), jnp.float32),
    mesh=_mesh,
    scratch_types=[
        pltpu.VMEM_SHARED((NPAD, FH), jnp.float32),
        pltpu.VMEM((NBUF, GI, CH), jnp.int32),
        pltpu.VMEM((NBUF, GI, CH), jnp.int32),
        pltpu.VMEM((BLK2, L), jnp.float32),
    ] + [pltpu.VMEM((CH, FH), jnp.float32)] * NBUF
      + [pltpu.SemaphoreType.DMA] * (4 * NBUF),
)
def _hops_kernel(g0_hbm, src3_hbm, dst3_hbm, i2_hbm, out_hbm,
                 acc, sidx, didx, i2v, *bs):
    bufs = bs[:NBUF]
    semg = bs[NBUF:2 * NBUF]
    semsi = bs[2 * NBUF:3 * NBUF]
    semdi = bs[3 * NBUF:4 * NBUF]
    c = lax.axis_index("c")
    s = lax.axis_index("s")
    g0half = g0_hbm.at[c]
    base = s * WRO

    @pl.loop(0, NBLK2)
    def _(t):
        pltpu.sync_copy(g0half.at[pl.ds(base + t * BLK2, BLK2)],
                        bufs[0].at[pl.ds(0, BLK2)])
        pltpu.sync_copy(bufs[0].at[pl.ds(0, BLK2)],
                        acc.at[pl.ds(base + t * BLK2, BLK2)])

    plsc.subcore_barrier()

    def fetch_idx(g, p):
        pltpu.async_copy(src3_hbm.at[s].at[pl.ds(g * GI, GI)], sidx.at[p],
                         semsi[p])
        pltpu.async_copy(dst3_hbm.at[s].at[pl.ds(g * GI, GI)], didx.at[p],
                         semdi[p])

    def wait_idx_s(p):
        pltpu.make_async_copy(src3_hbm.at[s].at[pl.ds(0, GI)], sidx.at[p],
                              semsi[p]).wait()

    def wait_idx_d(p):
        pltpu.make_async_copy(dst3_hbm.at[s].at[pl.ds(0, GI)], didx.at[p],
                              semdi[p]).wait()

    def gather_scatter(gsrc):
        def wait_gather(b):
            pltpu.make_async_copy(gsrc.at[pl.ds(0, CH)], bufs[b],
                                  semg[b]).wait()

        def start_gather(p, b, buf):
            pltpu.async_copy(gsrc.at[sidx.at[p, b]], bufs[buf], semg[buf])

        def group(g, p, is_last):
            if not is_last:
                fetch_idx(g + 1, 1 - p)
            for b in range(GI):
                bb = b % NBUF
                wait_gather(bb)
                pltpu.sync_copy(bufs[bb], acc.at[didx.at[p, b]], add=True)
                nb = b + NBUF
                if nb < GI:
                    start_gather(p, nb, bb)
                elif not is_last:
                    if nb == GI:
                        wait_idx_s(1 - p)
                    start_gather(1 - p, nb - GI, bb)
            if not is_last:
                wait_idx_d(1 - p)

        fetch_idx(0, 0)
        wait_idx_s(0)
        wait_idx_d(0)
        for b in range(NBUF):
            start_gather(0, b, b)

        @pl.loop(0, (NG - 1) // 2)
        def _(t):
            group(2 * t, 0, False)
            group(2 * t + 1, 1, False)

        group(NG - 1, 0, True)

    def scale(gdst):
        @pl.loop(0, NBLK2)
        def _(t):
            r0 = base + t * BLK2
            pltpu.sync_copy(i2_hbm.at[pl.ds(r0, BLK2)], i2v)
            pltpu.sync_copy(acc.at[pl.ds(r0, BLK2)],
                            bufs[0].at[pl.ds(0, BLK2)])

            @pl.loop(0, BLK2)
            def _(r):
                iv = i2v[r, pl.ds(0, L)]
                for cc in range(FH // L):
                    bufs[0][r, pl.ds(cc * L, L)] = (
                        bufs[0][r, pl.ds(cc * L, L)] * iv)

            pltpu.sync_copy(bufs[0].at[pl.ds(0, BLK2)],
                            gdst.at[pl.ds(r0, BLK2)])
            pltpu.sync_copy(bufs[0].at[pl.ds(0, BLK2)],
                            acc.at[pl.ds(r0, BLK2)])

    gather_scatter(g0half)
    plsc.subcore_barrier()
    scale(out_hbm.at[0].at[c])
    plsc.subcore_barrier()

    @pl.loop(0, HOPS - 1)
    def _(k):
        gather_scatter(out_hbm.at[k].at[c])
        plsc.subcore_barrier()
        scale(out_hbm.at[k + 1].at[c])
        plsc.subcore_barrier()


def _mm1(x, w0, b0):
    def body(x_ref, w_ref, b_ref, o_ref):
        o_ref[...] = jnp.maximum(
            jnp.dot(x_ref[...], w_ref[...],
                    preferred_element_type=jnp.float32) + b_ref[...], 0.0)

    return pl.pallas_call(
        body,
        grid=(N // RB,),
        in_specs=[
            pl.BlockSpec((RB, F), lambda i: (i, 0)),
            pl.BlockSpec((F, F), lambda i: (0, 0)),
            pl.BlockSpec((1, F), lambda i: (0, 0)),
        ],
        out_specs=pl.BlockSpec((RB, F), lambda i: (i, 0)),
        out_shape=jax.ShapeDtypeStruct((N, F), jnp.float32),
    )(x, w0, b0.reshape(1, F))


def _prep(cnt, h0):
    def body(c_ref, h_ref, g_ref, i2_ref, sq_ref):
        deg = c_ref[...] + 1.0
        isd = lax.rsqrt(deg)
        i2_ref[...] = jnp.broadcast_to(1.0 / deg, (RB, L))
        sq_ref[...] = jnp.sqrt(deg)
        h = h_ref[...]
        g_ref[0] = isd * h[:, :FH]
        g_ref[1] = isd * h[:, FH:]

    return pl.pallas_call(
        body,
        grid=(N // RB,),
        in_specs=[
            pl.BlockSpec((RB, 1), lambda i: (i, 0)),
            pl.BlockSpec((RB, F), lambda i: (i, 0)),
        ],
        out_specs=[
            pl.BlockSpec((NC, RB, FH), lambda i: (0, i, 0)),
            pl.BlockSpec((RB, L), lambda i: (i, 0)),
            pl.BlockSpec((RB, 1), lambda i: (i, 0)),
        ],
        out_shape=[
            jax.ShapeDtypeStruct((NC, N, FH), jnp.float32),
            jax.ShapeDtypeStruct((N, L), jnp.float32),
            jax.ShapeDtypeStruct((N, 1), jnp.float32),
        ],
    )(cnt, h0)


def _final(sqd, w_r, b, g0, gs):
    def body(sq_ref, w_ref, b_ref, g0_ref, gs_ref, o_ref):
        sq = sq_ref[...]
        acc = jnp.zeros((RB, F), jnp.float32) + b_ref[...]
        for h in range(NC):
            acc += jnp.dot(sq * g0_ref[h], w_ref[0, h],
                           preferred_element_type=jnp.float32)
        for k in range(HOPS):
            gk = gs_ref[k]
            for h in range(NC):
                acc += jnp.dot(sq * gk[h], w_ref[1 + k, h],
                               preferred_element_type=jnp.float32)
        o_ref[...] = jnp.maximum(acc, 0.0)

    return pl.pallas_call(
        body,
        grid=(N // RB,),
        in_specs=[
            pl.BlockSpec((RB, 1), lambda i: (i, 0)),
            pl.BlockSpec((1 + HOPS, NC, FH, F), lambda i: (0, 0, 0, 0)),
            pl.BlockSpec((1, F), lambda i: (0, 0)),
            pl.BlockSpec((NC, RB, FH), lambda i: (0, i, 0)),
            pl.BlockSpec((HOPS, NC, RB, FH), lambda i: (0, 0, i, 0)),
        ],
        out_specs=pl.BlockSpec((RB, F), lambda i: (i, 0)),
        out_shape=jax.ShapeDtypeStruct((N, F), jnp.float32),
    )(sqd, w_r, b, g0, gs)


def kernel(edge_index, features, W0, b0, W_rn, b_rn):
    src = edge_index[0].astype(jnp.int32)
    dst = edge_index[1].astype(jnp.int32)
    pad = EPAD - E
    prng = jnp.arange(pad, dtype=jnp.int32)
    src3 = jnp.concatenate(
        [src, (prng * 97) % N]).reshape(NS, NCH, CH)
    dst3 = jnp.concatenate(
        [dst, N + prng % (NPAD - N)]).reshape(NS, NCH, CH)

    degh = _deg_kernel(dst3)
    h0 = _mm1(features, W0, b0)
    g0, i2v, sqd = _prep(degh[:N, :1], h0)

    g0p = jnp.pad(g0, ((0, 0), (0, NPAD - N), (0, 0)))
    i2p = jnp.pad(i2v, ((0, NPAD - N), (0, 0)))
    gs = _hops_kernel(g0p, src3, dst3, i2p)

    w_r = W_rn.reshape(1 + HOPS, NC, FH, F)
    return _final(sqd, w_r, b_rn.reshape(1, F), g0, gs)


# speedup vs baseline: 1.0352x; 1.0352x over previous
import functools

import jax
import jax.numpy as jnp
from jax import lax
from jax.experimental import pallas as pl
from jax.experimental.pallas import tpu as pltpu
from jax.experimental.pallas import tpu_sc as plsc

N = 10000
E = 160000
F = 256
FH = 128
HOPS = 6
NC, NS, L = 2, 16, 16
CH = 128
NBUF = 2
NCH = 80
EPS = NCH * CH
EPAD = EPS * NS
NPAD = 10240
WRO = NPAD // NS
NBLK = WRO // CH
BLK2 = 64
NBLK2 = WRO // BLK2
RB = 1000

_mesh = plsc.VectorSubcoreMesh(
    core_axis_name="c", subcore_axis_name="s", num_cores=NC, num_subcores=NS)


def _fill(buf, rows, val):
    v = jnp.full((L,), val, jnp.float32)

    @pl.loop(0, rows)
    def _(r):
        for c in range(buf.shape[1] // L):
            buf[r, pl.ds(c * L, L)] = v


NCHH = NCH // 2


@functools.partial(
    pl.kernel,
    out_type=jax.ShapeDtypeStruct((NC, NPAD, L), jnp.float32),
    mesh=_mesh,
    scratch_types=[
        pltpu.VMEM_SHARED((NPAD, L), jnp.float32),
        pltpu.VMEM((NCHH, CH), jnp.int32),
        pltpu.VMEM((CH, L), jnp.float32),
    ],
)
def _deg_kernel(dst3_hbm, out_hbm, hist, didx, ones_v):
    c = lax.axis_index("c")
    s = lax.axis_index("s")

    _fill(ones_v, CH, 0.0)

    @pl.loop(0, NBLK)
    def _(t):
        pltpu.sync_copy(ones_v, hist.at[pl.ds(s * WRO + t * CH, CH)])

    _fill(ones_v, CH, 1.0)
    pltpu.sync_copy(dst3_hbm.at[s].at[pl.ds(c * NCHH, NCHH)], didx)
    plsc.subcore_barrier()

    @pl.loop(0, NCHH)
    def _(j):
        pltpu.sync_copy(ones_v, hist.at[didx.at[j]], add=True)

    plsc.subcore_barrier()
    pltpu.sync_copy(hist.at[pl.ds(s * WRO, WRO)],
                    out_hbm.at[c].at[pl.ds(s * WRO, WRO)])


GI = 8
NG = NCH // GI


@functools.partial(
    pl.kernel,
    out_type=jax.ShapeDtypeStruct((HOPS, NC, NPAD, FH), jnp.float32),
    mesh=_mesh,
    scratch_types=[
        pltpu.VMEM_SHARED((NPAD, FH), jnp.float32),
        pltpu.VMEM((2, GI, CH), jnp.int32),
        pltpu.VMEM((2, GI, CH), jnp.int32),
        pltpu.VMEM((BLK2, L), jnp.float32),
    ] + [pltpu.VMEM((CH, FH), jnp.float32)] * NBUF
      + [pltpu.SemaphoreType.DMA] * (4 * NBUF + 2),
)
def _hops_kernel(g0_hbm, src3_hbm, dst3_hbm, i2_hbm, out_hbm,
                 acc, sidx, didx, i2v, *bs):
    bufs = bs[:NBUF]
    semg = bs[NBUF:2 * NBUF]
    semsi = bs[2 * NBUF:3 * NBUF]
    semdi = bs[3 * NBUF:4 * NBUF]
    semwr = bs[4 * NBUF:4 * NBUF + 2]
    c = lax.axis_index("c")
    s = lax.axis_index("s")
    g0half = g0_hbm.at[c]
    base = s * WRO

    @pl.loop(0, NBLK2)
    def _(t):
        pltpu.sync_copy(g0half.at[pl.ds(base + t * BLK2, BLK2)],
                        bufs[0].at[pl.ds(0, BLK2)])
        pltpu.sync_copy(bufs[0].at[pl.ds(0, BLK2)],
                        acc.at[pl.ds(base + t * BLK2, BLK2)])

    plsc.subcore_barrier()

    def fetch_idx(g, p):
        pltpu.async_copy(src3_hbm.at[s].at[pl.ds(g * GI, GI)], sidx.at[p],
                         semsi[p])
        pltpu.async_copy(dst3_hbm.at[s].at[pl.ds(g * GI, GI)], didx.at[p],
                         semdi[p])

    def wait_idx_s(p):
        pltpu.make_async_copy(src3_hbm.at[s].at[pl.ds(0, GI)], sidx.at[p],
                              semsi[p]).wait()

    def wait_idx_d(p):
        pltpu.make_async_copy(dst3_hbm.at[s].at[pl.ds(0, GI)], didx.at[p],
                              semdi[p]).wait()

    def gather_scatter(gsrc):
        def wait_gather(b):
            pltpu.make_async_copy(gsrc.at[pl.ds(0, CH)], bufs[b],
                                  semg[b]).wait()

        def start_gather(p, b, buf):
            pltpu.async_copy(gsrc.at[sidx.at[p, b]], bufs[buf], semg[buf])

        def group(g, p, is_last):
            if not is_last:
                fetch_idx(g + 1, 1 - p)
            for b in range(GI):
                bb = b % NBUF
                wait_gather(bb)
                pltpu.sync_copy(bufs[bb], acc.at[didx.at[p, b]], add=True)
                nb = b + NBUF
                if nb < GI:
                    start_gather(p, nb, bb)
                elif not is_last:
                    if nb == GI:
                        wait_idx_s(1 - p)
                    start_gather(1 - p, nb - GI, bb)
            if not is_last:
                wait_idx_d(1 - p)

        fetch_idx(0, 0)
        wait_idx_s(0)
        wait_idx_d(0)
        for b in range(NBUF):
            start_gather(0, b, b)

        @pl.loop(0, (NG - 2) // 2)
        def _(t):
            group(2 * t, 0, False)
            group(2 * t + 1, 1, False)

        group(NG - 2, 0, False)
        group(NG - 1, 1, True)

    def scale(gdst):
        def wait_wr(sl):
            pltpu.make_async_copy(bufs[sl].at[pl.ds(0, BLK2)],
                                  gdst.at[pl.ds(0, BLK2)], semdi[sl]).wait()
            pltpu.make_async_copy(bufs[sl].at[pl.ds(0, BLK2)],
                                  acc.at[pl.ds(0, BLK2)], semwr[sl]).wait()

        @pl.loop(0, NBLK2 // 2)
        def _(u):
            for sl in (0, 1):
                tt = 2 * u + sl
                r0 = base + tt * BLK2
                pltpu.sync_copy(i2_hbm.at[pl.ds(r0, BLK2)], i2v)

                @pl.when(u > 0)
                def _():
                    wait_wr(sl)

                pltpu.sync_copy(acc.at[pl.ds(r0, BLK2)],
                                bufs[sl].at[pl.ds(0, BLK2)])

                @pl.loop(0, BLK2)
                def _(r):
                    iv = i2v[r, pl.ds(0, L)]
                    for cc in range(FH // L):
                        bufs[sl][r, pl.ds(cc * L, L)] = (
                            bufs[sl][r, pl.ds(cc * L, L)] * iv)

                pltpu.async_copy(bufs[sl].at[pl.ds(0, BLK2)],
                                 gdst.at[pl.ds(r0, BLK2)], semdi[sl])
                pltpu.async_copy(bufs[sl].at[pl.ds(0, BLK2)],
                                 acc.at[pl.ds(r0, BLK2)], semwr[sl])

        wait_wr(0)
        wait_wr(1)

    gather_scatter(g0half)
    plsc.subcore_barrier()
    scale(out_hbm.at[0].at[c])
    plsc.subcore_barrier()

    @pl.loop(0, HOPS - 1)
    def _(k):
        gather_scatter(out_hbm.at[k].at[c])
        plsc.subcore_barrier()
        scale(out_hbm.at[k + 1].at[c])
        plsc.subcore_barrier()


def _mm1(x, w0, b0):
    def body(x_ref, w_ref, b_ref, o_ref):
        o_ref[...] = jnp.maximum(
            jnp.dot(x_ref[...], w_ref[...],
                    preferred_element_type=jnp.float32) + b_ref[...], 0.0)

    return pl.pallas_call(
        body,
        grid=(N // RB,),
        in_specs=[
            pl.BlockSpec((RB, F), lambda i: (i, 0)),
            pl.BlockSpec((F, F), lambda i: (0, 0)),
            pl.BlockSpec((1, F), lambda i: (0, 0)),
        ],
        out_specs=pl.BlockSpec((RB, F), lambda i: (i, 0)),
        out_shape=jax.ShapeDtypeStruct((N, F), jnp.float32),
    )(x, w0, b0.reshape(1, F))


def _prep(cnt, h0):
    def body(c_ref, h_ref, g_ref, i2_ref, sq_ref):
        deg = c_ref[0] + c_ref[1] + 1.0
        isd = lax.rsqrt(deg)
        i2_ref[...] = jnp.broadcast_to(1.0 / deg, (RB, L))
        sq_ref[...] = jnp.sqrt(deg)
        h = h_ref[...]
        g_ref[0] = isd * h[:, :FH]
        g_ref[1] = isd * h[:, FH:]

    return pl.pallas_call(
        body,
        grid=(N // RB,),
        in_specs=[
            pl.BlockSpec((NC, RB, 1), lambda i: (0, i, 0)),
            pl.BlockSpec((RB, F), lambda i: (i, 0)),
        ],
        out_specs=[
            pl.BlockSpec((NC, RB, FH), lambda i: (0, i, 0)),
            pl.BlockSpec((RB, L), lambda i: (i, 0)),
            pl.BlockSpec((RB, 1), lambda i: (i, 0)),
        ],
        out_shape=[
            jax.ShapeDtypeStruct((NC, N, FH), jnp.float32),
            jax.ShapeDtypeStruct((N, L), jnp.float32),
            jax.ShapeDtypeStruct((N, 1), jnp.float32),
        ],
    )(cnt, h0)


def _final(sqd, w_r, b, g0, gs):
    def body(sq_ref, w_ref, b_ref, g0_ref, gs_ref, o_ref):
        sq = sq_ref[...]
        acc = jnp.zeros((RB, F), jnp.float32) + b_ref[...]
        for h in range(NC):
            acc += jnp.dot(sq * g0_ref[h], w_ref[0, h],
                           preferred_element_type=jnp.float32)
        for k in range(HOPS):
            gk = gs_ref[k]
            for h in range(NC):
                acc += jnp.dot(sq * gk[h], w_ref[1 + k, h],
                               preferred_element_type=jnp.float32)
        o_ref[...] = jnp.maximum(acc, 0.0)

    return pl.pallas_call(
        body,
        grid=(N // RB,),
        in_specs=[
            pl.BlockSpec((RB, 1), lambda i: (i, 0)),
            pl.BlockSpec((1 + HOPS, NC, FH, F), lambda i: (0, 0, 0, 0)),
            pl.BlockSpec((1, F), lambda i: (0, 0)),
            pl.BlockSpec((NC, RB, FH), lambda i: (0, i, 0)),
            pl.BlockSpec((HOPS, NC, RB, FH), lambda i: (0, 0, i, 0)),
        ],
        out_specs=pl.BlockSpec((RB, F), lambda i: (i, 0)),
        out_shape=jax.ShapeDtypeStruct((N, F), jnp.float32),
    )(sqd, w_r, b, g0, gs)


def kernel(edge_index, features, W0, b0, W_rn, b_rn):
    src = edge_index[0].astype(jnp.int32)
    dst = edge_index[1].astype(jnp.int32)
    pad = EPAD - E
    prng = jnp.arange(pad, dtype=jnp.int32)
    src3 = jnp.concatenate(
        [src, (prng * 97) % N]).reshape(NS, NCH, CH)
    dst3 = jnp.concatenate(
        [dst, N + prng % (NPAD - N)]).reshape(NS, NCH, CH)

    degh = _deg_kernel(dst3)
    h0 = _mm1(features, W0, b0)
    g0, i2v, sqd = _prep(degh[:, :N, :1], h0)

    g0p = jnp.pad(g0, ((0, 0), (0, NPAD - N), (0, 0)))
    i2p = jnp.pad(i2v, ((0, NPAD - N), (0, 0)))
    gs = _hops_kernel(g0p, src3, dst3, i2p)

    w_r = W_rn.reshape(1 + HOPS, NC, FH, F)
    return _final(sqd, w_r, b_rn.reshape(1, F), g0, gs)
